# baseline (speedup 1.0000x reference)
import functools
import jax
import jax.numpy as jnp
from jax import lax
from jax.experimental import pallas as pl
from jax.experimental.pallas import tpu as pltpu
from jax.experimental.pallas import tpu_sc as plsc

N_HASHES = 4
CHUNK = 144
RED = 4
PAD = 2360


def _embed_body(xp_ref, wt_ref, wa_ref, rotf_ref, bm_ref, ba_ref, comb_ref, codes_ref,
                *, BLK, Cin, C, D, H, W):
    HW = H * W
    base = pl.program_id(0) * BLK
    t = base + lax.broadcasted_iota(jnp.int32, (BLK, 1), 0)
    w_out = t % W
    h_out = (t // W) % H
    acc = jnp.zeros((BLK, C), dtype=jnp.float32)
    for k in range(27):
        kz, ky, kx = k // 9, (k // 3) % 3, k % 3
        off = (kz - 1) * HW + (ky - 1) * W + (kx - 1)
        sl = xp_ref[pl.ds(PAD + off + base, BLK), :]
        wv = w_out + (kx - 1)
        hv = h_out + (ky - 1)
        valid = (wv >= 0) & (wv < W) & (hv >= 0) & (hv < H)
        sl = jnp.where(valid, sl, 0.0)
        acc = acc + jax.lax.dot_general(sl, wt_ref[k], (((1,), (0,)), ((), ())),
                                        preferred_element_type=jnp.float32)
    acc = acc + bm_ref[:]
    xc = xp_ref[pl.ds(PAD + base, BLK), :]
    ya = jax.lax.dot_general(xc, wa_ref[:], (((1,), (0,)), ((), ())),
                             preferred_element_type=jnp.float32) + ba_ref[:]
    pad = jnp.zeros((BLK, 128 - C - Cin), dtype=jnp.float32)
    comb_ref[:] = jnp.concatenate([acc, ya, pad], axis=1)
    r = jax.lax.dot_general(acc, rotf_ref[:], (((1,), (0,)), ((), ())),
                            preferred_element_type=jnp.float32)
    nb2 = rotf_ref.shape[1] // N_HASHES
    iota64 = lax.broadcasted_iota(jnp.int32, (BLK, 2 * nb2), 1)
    cols = []
    for h in range(N_HASHES):
        rh = r[:, nb2 * h:nb2 * (h + 1)]
        both = jnp.concatenate([rh, -rh], axis=1)
        mx = jnp.max(both, axis=1, keepdims=True)
        idx = jnp.min(jnp.where(both >= mx, iota64, 2 * nb2), axis=1)
        cols.append(idx[:, None].astype(jnp.int32))
    codes_ref[:] = jnp.concatenate(
        cols + [jnp.zeros((BLK, 8 - N_HASHES), jnp.int32)], axis=1)


def _rank_body(codes_ref, tril_ref, g_ref, pref_ref, within_ref, *, RBLK, NB, NRB, L):
    h = pl.program_id(0)
    rb = pl.program_id(1)

    @pl.when(rb == 0)
    def _():
        pref_ref[:] = jnp.zeros((1, NB), jnp.float32)

    c = codes_ref[0, 0]
    onehot = (c == lax.broadcasted_iota(jnp.int32, (RBLK, NB), 1)).astype(jnp.float32)

    @pl.when(rb < NRB)
    def _():
        cum = jax.lax.dot_general(tril_ref[:], onehot, (((1,), (0,)), ((), ())),
                                  preferred_element_type=jnp.float32) + pref_ref[:]
        within_ref[rb] = jnp.sum(cum * onehot, axis=1, keepdims=True)
        pref_ref[:] = pref_ref[:] + jnp.sum(onehot, axis=0, keepdims=True)

    @pl.when(rb >= NRB)
    def _():
        ci = lax.broadcasted_iota(jnp.int32, (NB, NB), 0)
        cj = lax.broadcasted_iota(jnp.int32, (NB, NB), 1)
        ut = (ci < cj).astype(jnp.float32)
        offs = jax.lax.dot_general(pref_ref[:], ut, (((1,), (0,)), ((), ())),
                                   preferred_element_type=jnp.float32)
        offlook = jnp.sum(onehot * offs, axis=1, keepdims=True)
        g_ref[0, 0] = (within_ref[rb - NRB] + offlook).astype(jnp.int32) + h * L


def _attn_body(main_ref, left_ref, right_ref, out_ref, *, C, Cin, GB):
    left = left_ref[0, 0]
    right = right_ref[0, 0]
    for i in range(GB):
        cur = main_ref[0, i]
        prv = main_ref[0, i - 1] if i > 0 else left
        nxt = main_ref[0, i + 1] if i < GB - 1 else right
        xq = cur[:, :C]
        keys = jnp.concatenate([cur, prv, nxt], axis=0)
        xk = keys[:, :C]
        yv = keys[:, C:C + Cin]
        nrm = jnp.sqrt(jnp.sum(xk * xk, axis=1, keepdims=True))
        nrm = jnp.maximum(nrm, 5e-5)
        xkn = xk / nrm
        scores = jax.lax.dot_general(xq, xkn, (((1,), (1,)), ((), ())),
                                     preferred_element_type=jnp.float32,
                                     precision=jax.lax.Precision.HIGHEST)
        m = jnp.max(scores, axis=1, keepdims=True)
        e = jnp.exp(scores - m)
        s = jnp.sum(e, axis=1, keepdims=True)
        p = (e / s).astype(jnp.bfloat16)
        ret = jax.lax.dot_general(p, yv.astype(jnp.bfloat16),
                                  (((1,), (0,)), ((), ())),
                                  preferred_element_type=jnp.float32)
        b = m + jnp.log(s)
        out_ref[0, i] = jnp.concatenate(
            [ret, jnp.broadcast_to(b, (ret.shape[0], 128 - Cin))], axis=1)


def _combine_body(rt_ref, x_ref, out_ref, *, Cin, NPB, NCB):
    i = pl.program_id(0)
    out_ref[:] = jnp.zeros_like(out_ref)

    @pl.when((i >= NPB) & (i < NPB + NCB))
    def _():
        xs = [rt_ref[h, :, Cin:Cin + 1] for h in range(N_HASHES)]
        m = xs[0]
        for h in range(1, N_HASHES):
            m = jnp.maximum(m, xs[h])
        es = [jnp.exp(s - m) for s in xs]
        den = es[0]
        for h in range(1, N_HASHES):
            den = den + es[h]
        out = jnp.zeros_like(x_ref[:])
        for h in range(N_HASHES):
            out = out + rt_ref[h, :, :Cin] * (es[h] / den)
        out_ref[:] = out + x_ref[:]


def _final_body(xp_ref, wt_ref, bc_ref, out_ref, *, BLK, Cin, D, H, W, PADF):
    HW = H * W
    base = pl.program_id(0) * BLK
    t = base + lax.broadcasted_iota(jnp.int32, (BLK, 1), 0)
    w_out = t % W
    h_out = (t // W) % H
    acc = jnp.zeros((BLK, Cin), dtype=jnp.float32)
    for k in range(27):
        kz, ky, kx = k // 9, (k // 3) % 3, k % 3
        off = (kz - 1) * HW + (ky - 1) * W + (kx - 1)
        sl = jnp.maximum(xp_ref[pl.ds(PADF + off + base, BLK), :], 0.0)
        wv = w_out + (kx - 1)
        hv = h_out + (ky - 1)
        valid = (wv >= 0) & (wv < W) & (hv >= 0) & (hv < H)
        sl = jnp.where(valid, sl, 0.0)
        acc = acc + jax.lax.dot_general(sl, wt_ref[k], (((1,), (0,)), ((), ())),
                                        preferred_element_type=jnp.float32)
    out_ref[:] = xp_ref[pl.ds(PADF + base, BLK), :] + acc + bc_ref[:]


def _sc_scatter_body(comb_hbm, g_hbm, table_hbm, idx_v, src_v, sem, *, NC, L):
    wid = lax.axis_index("s") * NC + lax.axis_index("c")
    base_t = (wid % 8) * 1152
    pltpu.sync_copy(g_hbm.at[wid], idx_v)
    for j in range(9):
        pltpu.sync_copy(comb_hbm.at[pl.ds(base_t + j * 128, 128)], src_v)
        pltpu.async_copy(src_v, table_hbm.at[idx_v.at[j]], sem).wait()


def _sc_gather_body(table_hbm, g_hbm, out_hbm, idx_v, dst_v, sem, *, NC, L):
    wid = lax.axis_index("s") * NC + lax.axis_index("c")
    base = wid * 1152
    pltpu.sync_copy(g_hbm.at[wid], idx_v)
    for j in range(9):
        pltpu.async_copy(table_hbm.at[idx_v.at[j]], dst_v, sem).wait()
        pltpu.sync_copy(dst_v, out_hbm.at[pl.ds(base + j * 128, 128)])


def kernel(x, w_match, b_match, w_asm, b_asm, rot, w_conv, b_conv):
    N, Cin, D, H, W = x.shape
    L = D * H * W
    C = Cin // RED
    NK = L // CHUNK
    xt = x.reshape(Cin, L).T
    xp = jnp.zeros((L + 2 * PAD, Cin), jnp.float32).at[PAD:PAD + L].set(xt)
    wt = w_match.transpose(2, 3, 4, 1, 0).reshape(27, Cin, C)
    wa = w_asm.reshape(Cin, Cin).T
    rotf = rot.reshape(C, -1)
    wc = w_conv.transpose(2, 3, 4, 1, 0).reshape(27, Cin, Cin)

    EBLK = 1152
    comb, codes8 = pl.pallas_call(
        functools.partial(_embed_body, BLK=EBLK, Cin=Cin, C=C, D=D, H=H, W=W),
        grid=(L // EBLK,),
        in_specs=[
            pl.BlockSpec((L + 2 * PAD, Cin), lambda i: (0, 0)),
            pl.BlockSpec((27, Cin, C), lambda i: (0, 0, 0)),
            pl.BlockSpec((Cin, Cin), lambda i: (0, 0)),
            pl.BlockSpec((C, 128), lambda i: (0, 0)),
            pl.BlockSpec((1, C), lambda i: (0, 0)),
            pl.BlockSpec((1, Cin), lambda i: (0, 0)),
        ],
        out_specs=[pl.BlockSpec((EBLK, 128), lambda i: (i, 0)),
                   pl.BlockSpec((EBLK, 8), lambda i: (i, 0))],
        out_shape=[jax.ShapeDtypeStruct((L, 128), jnp.float32),
                   jax.ShapeDtypeStruct((L, 8), jnp.int32)],
    )(xp, wt, wa, rotf, b_match.reshape(1, C), b_asm.reshape(1, Cin))

    RBLK = 1152
    NRB = L // RBLK
    NB = 64
    codes3 = codes8[:, :N_HASHES].T.reshape(N_HASHES, NRB, RBLK, 1)
    tril = jnp.tril(jnp.ones((RBLK, RBLK), jnp.float32), -1)
    g4 = pl.pallas_call(
        functools.partial(_rank_body, RBLK=RBLK, NB=NB, NRB=NRB, L=L),
        grid=(N_HASHES, 2 * NRB),
        in_specs=[
            pl.BlockSpec((1, 1, RBLK, 1), lambda h, rb: (h, rb % NRB, 0, 0)),
            pl.BlockSpec((RBLK, RBLK), lambda h, rb: (0, 0)),
        ],
        out_specs=pl.BlockSpec((1, 1, RBLK, 1), lambda h, rb: (h, rb % NRB, 0, 0)),
        out_shape=jax.ShapeDtypeStruct((N_HASHES, NRB, RBLK, 1), jnp.int32),
        scratch_shapes=[pltpu.VMEM((1, NB), jnp.float32),
                        pltpu.VMEM((NRB, RBLK, 1), jnp.float32)],
    )(codes3, tril)
    g32 = g4.reshape(32, 9, 128)

    mesh = plsc.VectorSubcoreMesh(core_axis_name="c", subcore_axis_name="s")
    table = pl.kernel(
        functools.partial(_sc_scatter_body, NC=2, L=L),
        mesh=mesh,
        out_type=jax.ShapeDtypeStruct((N_HASHES * L, 128), jnp.float32),
        scratch_types=[pltpu.VMEM((9, 128), jnp.int32),
                       pltpu.VMEM((128, 128), jnp.float32),
                       pltpu.SemaphoreType.DMA],
    )(comb, g32)
    table = table.reshape(N_HASHES, NK, CHUNK, 128)

    GB = 8
    ret_plus = pl.pallas_call(
        functools.partial(_attn_body, C=C, Cin=Cin, GB=GB),
        grid=(N_HASHES, NK // GB),
        in_specs=[
            pl.BlockSpec((1, GB, CHUNK, 128), lambda h, kb: (h, kb, 0, 0)),
            pl.BlockSpec((1, 1, CHUNK, 128),
                         lambda h, kb: (h, (kb * GB - 1) % NK, 0, 0)),
            pl.BlockSpec((1, 1, CHUNK, 128),
                         lambda h, kb: (h, ((kb + 1) * GB) % NK, 0, 0)),
        ],
        out_specs=pl.BlockSpec((1, GB, CHUNK, 128), lambda h, kb: (h, kb, 0, 0)),
        out_shape=jax.ShapeDtypeStruct((N_HASHES, NK, CHUNK, 128), jnp.float32),
    )(table, table, table)

    rt = pl.kernel(
        functools.partial(_sc_gather_body, NC=2, L=L),
        mesh=mesh,
        out_type=jax.ShapeDtypeStruct((N_HASHES * L, 128), jnp.float32),
        scratch_types=[pltpu.VMEM((9, 128), jnp.int32),
                       pltpu.VMEM((128, 128), jnp.float32),
                       pltpu.SemaphoreType.DMA],
    )(ret_plus.reshape(N_HASHES * L, 128), g32)
    rt = rt.reshape(N_HASHES, L, 128)

    BLK = 1152
    PAD2 = 3 * BLK
    NCB = L // BLK
    xp1 = pl.pallas_call(
        functools.partial(_combine_body, Cin=Cin, NPB=3, NCB=NCB),
        grid=(NCB + 6,),
        in_specs=[
            pl.BlockSpec((N_HASHES, BLK, 128), lambda i: (0, (i - 3) % NCB, 0)),
            pl.BlockSpec((BLK, Cin), lambda i: ((i - 3) % NCB, 0)),
        ],
        out_specs=pl.BlockSpec((BLK, Cin), lambda i: (i, 0)),
        out_shape=jax.ShapeDtypeStruct((L + 2 * PAD2, Cin), jnp.float32),
    )(rt, xt)

    out = pl.pallas_call(
        functools.partial(_final_body, BLK=BLK, Cin=Cin, D=D, H=H, W=W, PADF=PAD2),
        grid=(L // BLK,),
        in_specs=[
            pl.BlockSpec((L + 2 * PAD2, Cin), lambda i: (0, 0)),
            pl.BlockSpec((27, Cin, Cin), lambda i: (0, 0, 0)),
            pl.BlockSpec((1, Cin), lambda i: (0, 0)),
        ],
        out_specs=pl.BlockSpec((BLK, Cin), lambda i: (i, 0)),
        out_shape=jax.ShapeDtypeStruct((L, Cin), jnp.float32),
    )(xp1, wc, b_conv.reshape(1, Cin))

    return out.T.reshape(N, Cin, D, H, W)


# speedup vs baseline: 1.1069x; 1.1069x over previous
import functools
import jax
import jax.numpy as jnp
from jax import lax
from jax.experimental import pallas as pl
from jax.experimental.pallas import tpu as pltpu
from jax.experimental.pallas import tpu_sc as plsc

N_HASHES = 4
CHUNK = 144
RED = 4
PAD = 2360


def _embed_body(xp_ref, wt_ref, wa_ref, rotf_ref, bm_ref, ba_ref, comb_ref, codes_ref,
                *, BLK, Cin, C, D, H, W):
    HW = H * W
    base = pl.program_id(0) * BLK
    t = base + lax.broadcasted_iota(jnp.int32, (BLK, 1), 0)
    w_out = t % W
    h_out = (t // W) % H
    acc = jnp.zeros((BLK, C), dtype=jnp.float32)
    for k in range(27):
        kz, ky, kx = k // 9, (k // 3) % 3, k % 3
        off = (kz - 1) * HW + (ky - 1) * W + (kx - 1)
        sl = xp_ref[pl.ds(PAD + off + base, BLK), :]
        wv = w_out + (kx - 1)
        hv = h_out + (ky - 1)
        valid = (wv >= 0) & (wv < W) & (hv >= 0) & (hv < H)
        sl = jnp.where(valid, sl, 0.0)
        acc = acc + jax.lax.dot_general(sl, wt_ref[k], (((1,), (0,)), ((), ())),
                                        preferred_element_type=jnp.float32)
    acc = acc + bm_ref[:]
    xc = xp_ref[pl.ds(PAD + base, BLK), :]
    ya = jax.lax.dot_general(xc, wa_ref[:], (((1,), (0,)), ((), ())),
                             preferred_element_type=jnp.float32) + ba_ref[:]
    pad = jnp.zeros((BLK, 128 - C - Cin), dtype=jnp.float32)
    comb_ref[:] = jnp.concatenate([acc, ya, pad], axis=1)
    r = jax.lax.dot_general(acc, rotf_ref[:], (((1,), (0,)), ((), ())),
                            preferred_element_type=jnp.float32)
    nb2 = rotf_ref.shape[1] // N_HASHES
    iota64 = lax.broadcasted_iota(jnp.int32, (BLK, 2 * nb2), 1)
    cols = []
    for h in range(N_HASHES):
        rh = r[:, nb2 * h:nb2 * (h + 1)]
        both = jnp.concatenate([rh, -rh], axis=1)
        mx = jnp.max(both, axis=1, keepdims=True)
        idx = jnp.min(jnp.where(both >= mx, iota64, 2 * nb2), axis=1)
        cols.append(idx[:, None].astype(jnp.int32))
    codes_ref[:] = jnp.concatenate(
        cols + [jnp.zeros((BLK, 8 - N_HASHES), jnp.int32)], axis=1)


def _rank1_body(codes_ref, tril_ref, within_ref, counts_ref, pref_ref, *, RBLK, NB):
    rb = pl.program_id(1)

    @pl.when(rb == 0)
    def _():
        pref_ref[:] = jnp.zeros((1, NB), jnp.float32)

    c = codes_ref[0, 0]
    onehot = (c == lax.broadcasted_iota(jnp.int32, (RBLK, NB), 1)).astype(jnp.float32)
    cum = jax.lax.dot_general(tril_ref[:], onehot, (((1,), (0,)), ((), ())),
                              preferred_element_type=jnp.float32) + pref_ref[:]
    within_ref[0, 0] = jnp.sum(cum * onehot, axis=1, keepdims=True)
    pref_ref[:] = pref_ref[:] + jnp.sum(onehot, axis=0, keepdims=True)
    counts_ref[0] = pref_ref[:]


def _rank2_body(codes_ref, within_ref, counts_ref, g_ref, *, RBLK, NB, L):
    h = pl.program_id(0)
    c = codes_ref[0, 0]
    onehot = (c == lax.broadcasted_iota(jnp.int32, (RBLK, NB), 1)).astype(jnp.float32)
    ci = lax.broadcasted_iota(jnp.int32, (NB, NB), 0)
    cj = lax.broadcasted_iota(jnp.int32, (NB, NB), 1)
    ut = (ci < cj).astype(jnp.float32)
    offs = jax.lax.dot_general(counts_ref[0], ut, (((1,), (0,)), ((), ())),
                               preferred_element_type=jnp.float32)
    offlook = jnp.sum(onehot * offs, axis=1, keepdims=True)
    g_ref[0, 0] = (within_ref[0, 0] + offlook).astype(jnp.int32) + h * L


def _attn_body(main_ref, left_ref, right_ref, out_ref, *, C, Cin, GB):
    left = left_ref[0, 0]
    right = right_ref[0, 0]
    for i in range(GB):
        cur = main_ref[0, i]
        prv = main_ref[0, i - 1] if i > 0 else left
        nxt = main_ref[0, i + 1] if i < GB - 1 else right
        xq = cur[:, :C]
        keys = jnp.concatenate([cur, prv, nxt], axis=0)
        xk = keys[:, :C]
        yv = keys[:, C:C + Cin]
        nrm = jnp.sqrt(jnp.sum(xk * xk, axis=1, keepdims=True))
        nrm = jnp.maximum(nrm, 5e-5)
        xkn = xk / nrm
        scores = jax.lax.dot_general(xq, xkn, (((1,), (1,)), ((), ())),
                                     preferred_element_type=jnp.float32)
        m = jnp.max(scores, axis=1, keepdims=True)
        e = jnp.exp(scores - m)
        s = jnp.sum(e, axis=1, keepdims=True)
        p = (e / s).astype(jnp.bfloat16)
        ret = jax.lax.dot_general(p, yv.astype(jnp.bfloat16),
                                  (((1,), (0,)), ((), ())),
                                  preferred_element_type=jnp.float32)
        b = m + jnp.log(s)
        out_ref[0, i] = jnp.concatenate(
            [ret, jnp.broadcast_to(b, (ret.shape[0], 128 - Cin))], axis=1)


def _combine_body(rt_ref, x_ref, out_ref, *, Cin, NPB, NCB):
    i = pl.program_id(0)
    out_ref[:] = jnp.zeros_like(out_ref)

    @pl.when((i >= NPB) & (i < NPB + NCB))
    def _():
        xs = [rt_ref[h, :, Cin:Cin + 1] for h in range(N_HASHES)]
        m = xs[0]
        for h in range(1, N_HASHES):
            m = jnp.maximum(m, xs[h])
        es = [jnp.exp(s - m) for s in xs]
        den = es[0]
        for h in range(1, N_HASHES):
            den = den + es[h]
        out = jnp.zeros_like(x_ref[:])
        for h in range(N_HASHES):
            out = out + rt_ref[h, :, :Cin] * (es[h] / den)
        out_ref[:] = out + x_ref[:]


def _final_body(xp_ref, wt_ref, bc_ref, out_ref, *, BLK, Cin, D, H, W, PADF):
    HW = H * W
    base = pl.program_id(0) * BLK
    t = base + lax.broadcasted_iota(jnp.int32, (BLK, 1), 0)
    w_out = t % W
    h_out = (t // W) % H
    acc = jnp.zeros((BLK, Cin), dtype=jnp.float32)
    for k in range(27):
        kz, ky, kx = k // 9, (k // 3) % 3, k % 3
        off = (kz - 1) * HW + (ky - 1) * W + (kx - 1)
        sl = jnp.maximum(xp_ref[pl.ds(PADF + off + base, BLK), :], 0.0)
        wv = w_out + (kx - 1)
        hv = h_out + (ky - 1)
        valid = (wv >= 0) & (wv < W) & (hv >= 0) & (hv < H)
        sl = jnp.where(valid, sl, 0.0)
        acc = acc + jax.lax.dot_general(sl, wt_ref[k], (((1,), (0,)), ((), ())),
                                        preferred_element_type=jnp.float32)
    out_ref[:] = xp_ref[pl.ds(PADF + base, BLK), :] + acc + bc_ref[:]


def _sc_scatter_body(comb_hbm, g_hbm, table_hbm, idx_v, src_v, sem, *, NC, L):
    wid = lax.axis_index("s") * NC + lax.axis_index("c")
    base_t = (wid % 8) * 1152
    pltpu.sync_copy(g_hbm.at[wid], idx_v)
    for j in range(9):
        pltpu.sync_copy(comb_hbm.at[pl.ds(base_t + j * 128, 128)], src_v)
        pltpu.async_copy(src_v, table_hbm.at[idx_v.at[j]], sem).wait()


def _sc_gather_body(table_hbm, g_hbm, out_hbm, idx_v, dst_v, sem, *, NC, L):
    wid = lax.axis_index("s") * NC + lax.axis_index("c")
    base = wid * 1152
    pltpu.sync_copy(g_hbm.at[wid], idx_v)
    for j in range(9):
        pltpu.async_copy(table_hbm.at[idx_v.at[j]], dst_v, sem).wait()
        pltpu.sync_copy(dst_v, out_hbm.at[pl.ds(base + j * 128, 128)])


def kernel(x, w_match, b_match, w_asm, b_asm, rot, w_conv, b_conv):
    N, Cin, D, H, W = x.shape
    L = D * H * W
    C = Cin // RED
    NK = L // CHUNK
    xt = x.reshape(Cin, L).T
    xp = jnp.zeros((L + 2 * PAD, Cin), jnp.float32).at[PAD:PAD + L].set(xt)
    wt = w_match.transpose(2, 3, 4, 1, 0).reshape(27, Cin, C)
    wa = w_asm.reshape(Cin, Cin).T
    rotf = rot.reshape(C, -1)
    wc = w_conv.transpose(2, 3, 4, 1, 0).reshape(27, Cin, Cin)

    EBLK = 1152
    comb, codes8 = pl.pallas_call(
        functools.partial(_embed_body, BLK=EBLK, Cin=Cin, C=C, D=D, H=H, W=W),
        grid=(L // EBLK,),
        in_specs=[
            pl.BlockSpec((L + 2 * PAD, Cin), lambda i: (0, 0)),
            pl.BlockSpec((27, Cin, C), lambda i: (0, 0, 0)),
            pl.BlockSpec((Cin, Cin), lambda i: (0, 0)),
            pl.BlockSpec((C, 128), lambda i: (0, 0)),
            pl.BlockSpec((1, C), lambda i: (0, 0)),
            pl.BlockSpec((1, Cin), lambda i: (0, 0)),
        ],
        out_specs=[pl.BlockSpec((EBLK, 128), lambda i: (i, 0)),
                   pl.BlockSpec((EBLK, 8), lambda i: (i, 0))],
        out_shape=[jax.ShapeDtypeStruct((L, 128), jnp.float32),
                   jax.ShapeDtypeStruct((L, 8), jnp.int32)],
    )(xp, wt, wa, rotf, b_match.reshape(1, C), b_asm.reshape(1, Cin))

    RBLK = 1152
    NRB = L // RBLK
    NB = 64
    codes3 = codes8[:, :N_HASHES].T.reshape(N_HASHES, NRB, RBLK, 1)
    tril = jnp.tril(jnp.ones((RBLK, RBLK), jnp.float32), -1)
    within, counts = pl.pallas_call(
        functools.partial(_rank1_body, RBLK=RBLK, NB=NB),
        grid=(N_HASHES, NRB),
        in_specs=[
            pl.BlockSpec((1, 1, RBLK, 1), lambda h, rb: (h, rb, 0, 0)),
            pl.BlockSpec((RBLK, RBLK), lambda h, rb: (0, 0)),
        ],
        out_specs=[pl.BlockSpec((1, 1, RBLK, 1), lambda h, rb: (h, rb, 0, 0)),
                   pl.BlockSpec((1, 1, NB), lambda h, rb: (h, 0, 0))],
        out_shape=[jax.ShapeDtypeStruct((N_HASHES, NRB, RBLK, 1), jnp.float32),
                   jax.ShapeDtypeStruct((N_HASHES, 1, NB), jnp.float32)],
        scratch_shapes=[pltpu.VMEM((1, NB), jnp.float32)],
    )(codes3, tril)
    g4 = pl.pallas_call(
        functools.partial(_rank2_body, RBLK=RBLK, NB=NB, L=L),
        grid=(N_HASHES, NRB),
        in_specs=[
            pl.BlockSpec((1, 1, RBLK, 1), lambda h, rb: (h, rb, 0, 0)),
            pl.BlockSpec((1, 1, RBLK, 1), lambda h, rb: (h, rb, 0, 0)),
            pl.BlockSpec((1, 1, NB), lambda h, rb: (h, 0, 0)),
        ],
        out_specs=pl.BlockSpec((1, 1, RBLK, 1), lambda h, rb: (h, rb, 0, 0)),
        out_shape=jax.ShapeDtypeStruct((N_HASHES, NRB, RBLK, 1), jnp.int32),
    )(codes3, within, counts)
    g32 = g4.reshape(32, 9, 128)

    mesh = plsc.VectorSubcoreMesh(core_axis_name="c", subcore_axis_name="s")
    table = pl.kernel(
        functools.partial(_sc_scatter_body, NC=2, L=L),
        mesh=mesh,
        out_type=jax.ShapeDtypeStruct((N_HASHES * L, 128), jnp.float32),
        scratch_types=[pltpu.VMEM((9, 128), jnp.int32),
                       pltpu.VMEM((128, 128), jnp.float32),
                       pltpu.SemaphoreType.DMA],
    )(comb, g32)
    table = table.reshape(N_HASHES, NK, CHUNK, 128)

    GB = 8
    ret_plus = pl.pallas_call(
        functools.partial(_attn_body, C=C, Cin=Cin, GB=GB),
        grid=(N_HASHES, NK // GB),
        in_specs=[
            pl.BlockSpec((1, GB, CHUNK, 128), lambda h, kb: (h, kb, 0, 0)),
            pl.BlockSpec((1, 1, CHUNK, 128),
                         lambda h, kb: (h, (kb * GB - 1) % NK, 0, 0)),
            pl.BlockSpec((1, 1, CHUNK, 128),
                         lambda h, kb: (h, ((kb + 1) * GB) % NK, 0, 0)),
        ],
        out_specs=pl.BlockSpec((1, GB, CHUNK, 128), lambda h, kb: (h, kb, 0, 0)),
        out_shape=jax.ShapeDtypeStruct((N_HASHES, NK, CHUNK, 128), jnp.float32),
    )(table, table, table)

    rt = pl.kernel(
        functools.partial(_sc_gather_body, NC=2, L=L),
        mesh=mesh,
        out_type=jax.ShapeDtypeStruct((N_HASHES * L, 128), jnp.float32),
        scratch_types=[pltpu.VMEM((9, 128), jnp.int32),
                       pltpu.VMEM((128, 128), jnp.float32),
                       pltpu.SemaphoreType.DMA],
    )(ret_plus.reshape(N_HASHES * L, 128), g32)
    rt = rt.reshape(N_HASHES, L, 128)

    BLK = 1152
    PAD2 = 3 * BLK
    NCB = L // BLK
    xp1 = pl.pallas_call(
        functools.partial(_combine_body, Cin=Cin, NPB=3, NCB=NCB),
        grid=(NCB + 6,),
        in_specs=[
            pl.BlockSpec((N_HASHES, BLK, 128), lambda i: (0, (i - 3) % NCB, 0)),
            pl.BlockSpec((BLK, Cin), lambda i: ((i - 3) % NCB, 0)),
        ],
        out_specs=pl.BlockSpec((BLK, Cin), lambda i: (i, 0)),
        out_shape=jax.ShapeDtypeStruct((L + 2 * PAD2, Cin), jnp.float32),
    )(rt, xt)

    out = pl.pallas_call(
        functools.partial(_final_body, BLK=BLK, Cin=Cin, D=D, H=H, W=W, PADF=PAD2),
        grid=(L // BLK,),
        in_specs=[
            pl.BlockSpec((L + 2 * PAD2, Cin), lambda i: (0, 0)),
            pl.BlockSpec((27, Cin, Cin), lambda i: (0, 0, 0)),
            pl.BlockSpec((1, Cin), lambda i: (0, 0)),
        ],
        out_specs=pl.BlockSpec((BLK, Cin), lambda i: (i, 0)),
        out_shape=jax.ShapeDtypeStruct((L, Cin), jnp.float32),
    )(xp1, wc, b_conv.reshape(1, Cin))

    return out.T.reshape(N, Cin, D, H, W)


# speedup vs baseline: 1.1180x; 1.0100x over previous
import functools
import jax
import jax.numpy as jnp
from jax import lax
from jax.experimental import pallas as pl
from jax.experimental.pallas import tpu as pltpu
from jax.experimental.pallas import tpu_sc as plsc

N_HASHES = 4
CHUNK = 144
RED = 4
PAD = 2360


def _embed_body(xp_ref, wt_ref, wa_ref, rotf_ref, bm_ref, ba_ref, comb_ref, codes_ref,
                *, BLK, Cin, C, D, H, W):
    HW = H * W
    base = pl.program_id(0) * BLK
    t = base + lax.broadcasted_iota(jnp.int32, (BLK, 1), 0)
    w_out = t % W
    h_out = (t // W) % H
    acc = jnp.zeros((BLK, C), dtype=jnp.float32)
    for k in range(27):
        kz, ky, kx = k // 9, (k // 3) % 3, k % 3
        off = (kz - 1) * HW + (ky - 1) * W + (kx - 1)
        sl = xp_ref[pl.ds(PAD + off + base, BLK), :]
        wv = w_out + (kx - 1)
        hv = h_out + (ky - 1)
        valid = (wv >= 0) & (wv < W) & (hv >= 0) & (hv < H)
        sl = jnp.where(valid, sl, 0.0)
        acc = acc + jax.lax.dot_general(sl, wt_ref[k], (((1,), (0,)), ((), ())),
                                        preferred_element_type=jnp.float32)
    acc = acc + bm_ref[:]
    xc = xp_ref[pl.ds(PAD + base, BLK), :]
    ya = jax.lax.dot_general(xc, wa_ref[:], (((1,), (0,)), ((), ())),
                             preferred_element_type=jnp.float32) + ba_ref[:]
    pad = jnp.zeros((BLK, 128 - C - Cin), dtype=jnp.float32)
    comb_ref[:] = jnp.concatenate([acc, ya, pad], axis=1)
    r = jax.lax.dot_general(acc, rotf_ref[:], (((1,), (0,)), ((), ())),
                            preferred_element_type=jnp.float32)
    nb2 = rotf_ref.shape[1] // N_HASHES
    iota64 = lax.broadcasted_iota(jnp.int32, (BLK, 2 * nb2), 1)
    cols = []
    for h in range(N_HASHES):
        rh = r[:, nb2 * h:nb2 * (h + 1)]
        both = jnp.concatenate([rh, -rh], axis=1)
        mx = jnp.max(both, axis=1, keepdims=True)
        idx = jnp.min(jnp.where(both >= mx, iota64, 2 * nb2), axis=1)
        cols.append(idx[:, None].astype(jnp.int32))
    codes_ref[:] = jnp.concatenate(
        cols + [jnp.zeros((BLK, 8 - N_HASHES), jnp.int32)], axis=1)


def _rank1_body(codes_ref, tril_ref, within_ref, counts_ref, pref_ref, *, RBLK, NB):
    rb = pl.program_id(1)

    @pl.when(rb == 0)
    def _():
        pref_ref[:] = jnp.zeros((1, NB), jnp.float32)

    c = codes_ref[0, 0]
    onehot = (c == lax.broadcasted_iota(jnp.int32, (RBLK, NB), 1)).astype(jnp.float32)
    cum = jax.lax.dot_general(tril_ref[:], onehot, (((1,), (0,)), ((), ())),
                              preferred_element_type=jnp.float32) + pref_ref[:]
    within_ref[0, 0] = jnp.sum(cum * onehot, axis=1, keepdims=True)
    pref_ref[:] = pref_ref[:] + jnp.sum(onehot, axis=0, keepdims=True)
    counts_ref[0] = pref_ref[:]


def _rank2_body(codes_ref, within_ref, counts_ref, g_ref, *, RBLK, NB, L):
    h = pl.program_id(0)
    c = codes_ref[0, 0]
    onehot = (c == lax.broadcasted_iota(jnp.int32, (RBLK, NB), 1)).astype(jnp.float32)
    ci = lax.broadcasted_iota(jnp.int32, (NB, NB), 0)
    cj = lax.broadcasted_iota(jnp.int32, (NB, NB), 1)
    ut = (ci < cj).astype(jnp.float32)
    offs = jax.lax.dot_general(counts_ref[0], ut, (((1,), (0,)), ((), ())),
                               preferred_element_type=jnp.float32)
    offlook = jnp.sum(onehot * offs, axis=1, keepdims=True)
    g_ref[0, 0] = (within_ref[0, 0] + offlook).astype(jnp.int32) + h * L


def _attn_body(main_ref, left_ref, right_ref, out_ref, *, C, Cin, GB):
    left = left_ref[0, 0]
    right = right_ref[0, 0]
    for i in range(GB):
        cur = main_ref[0, i]
        prv = main_ref[0, i - 1] if i > 0 else left
        nxt = main_ref[0, i + 1] if i < GB - 1 else right
        xq = cur[:, :C]
        keys = jnp.concatenate([cur, prv, nxt], axis=0)
        xk = keys[:, :C]
        yv = keys[:, C:C + Cin]
        nrm = jnp.sqrt(jnp.sum(xk * xk, axis=1, keepdims=True))
        nrm = jnp.maximum(nrm, 5e-5)
        xkn = xk / nrm
        scores = jax.lax.dot_general(xq, xkn, (((1,), (1,)), ((), ())),
                                     preferred_element_type=jnp.float32)
        m = jnp.max(scores, axis=1, keepdims=True)
        e = jnp.exp(scores - m)
        s = jnp.sum(e, axis=1, keepdims=True)
        p = e / s
        ret = jax.lax.dot_general(p, yv, (((1,), (0,)), ((), ())),
                                  preferred_element_type=jnp.float32)
        b = m + jnp.log(s)
        out_ref[0, i] = jnp.concatenate(
            [ret, jnp.broadcast_to(b, (ret.shape[0], 128 - Cin))], axis=1)


def _combine_body(rt_ref, x_ref, out_ref, *, Cin, NPB, NCB):
    i = pl.program_id(0)
    out_ref[:] = jnp.zeros_like(out_ref)

    @pl.when((i >= NPB) & (i < NPB + NCB))
    def _():
        xs = [rt_ref[h, :, Cin:Cin + 1] for h in range(N_HASHES)]
        m = xs[0]
        for h in range(1, N_HASHES):
            m = jnp.maximum(m, xs[h])
        es = [jnp.exp(s - m) for s in xs]
        den = es[0]
        for h in range(1, N_HASHES):
            den = den + es[h]
        out = jnp.zeros_like(x_ref[:])
        for h in range(N_HASHES):
            out = out + rt_ref[h, :, :Cin] * (es[h] / den)
        out_ref[:] = out + x_ref[:]


def _final_body(xp_ref, wt_ref, bc_ref, out_ref, *, BLK, Cin, D, H, W, PADF):
    HW = H * W
    base = pl.program_id(0) * BLK
    t = base + lax.broadcasted_iota(jnp.int32, (BLK, 1), 0)
    w_out = t % W
    h_out = (t // W) % H
    acc = jnp.zeros((BLK, Cin), dtype=jnp.float32)
    for k in range(27):
        kz, ky, kx = k // 9, (k // 3) % 3, k % 3
        off = (kz - 1) * HW + (ky - 1) * W + (kx - 1)
        sl = jnp.maximum(xp_ref[pl.ds(PADF + off + base, BLK), :], 0.0)
        wv = w_out + (kx - 1)
        hv = h_out + (ky - 1)
        valid = (wv >= 0) & (wv < W) & (hv >= 0) & (hv < H)
        sl = jnp.where(valid, sl, 0.0)
        acc = acc + jax.lax.dot_general(sl, wt_ref[k], (((1,), (0,)), ((), ())),
                                        preferred_element_type=jnp.float32)
    out_ref[:] = xp_ref[pl.ds(PADF + base, BLK), :] + acc + bc_ref[:]


def _sc_scatter_body(comb_hbm, g_hbm, table_hbm, idx_v, src_v, sem, *, NC, L):
    wid = lax.axis_index("s") * NC + lax.axis_index("c")
    base_t = (wid % 8) * 1152
    pltpu.sync_copy(g_hbm.at[wid], idx_v)
    for j in range(9):
        pltpu.sync_copy(comb_hbm.at[pl.ds(base_t + j * 128, 128)], src_v)
        pltpu.async_copy(src_v, table_hbm.at[idx_v.at[j]], sem).wait()


def _sc_gather_body(table_hbm, g_hbm, out_hbm, idx_v, dst_v, sem, *, NC, L):
    wid = lax.axis_index("s") * NC + lax.axis_index("c")
    base = wid * 1152
    pltpu.sync_copy(g_hbm.at[wid], idx_v)
    for j in range(9):
        pltpu.async_copy(table_hbm.at[idx_v.at[j]], dst_v, sem).wait()
        pltpu.sync_copy(dst_v, out_hbm.at[pl.ds(base + j * 128, 128)])


def kernel(x, w_match, b_match, w_asm, b_asm, rot, w_conv, b_conv):
    N, Cin, D, H, W = x.shape
    L = D * H * W
    C = Cin // RED
    NK = L // CHUNK
    xt = x.reshape(Cin, L).T
    xp = jnp.zeros((L + 2 * PAD, Cin), jnp.float32).at[PAD:PAD + L].set(xt)
    wt = w_match.transpose(2, 3, 4, 1, 0).reshape(27, Cin, C)
    wa = w_asm.reshape(Cin, Cin).T
    rotf = rot.reshape(C, -1)
    wc = w_conv.transpose(2, 3, 4, 1, 0).reshape(27, Cin, Cin)

    EBLK = 1152
    comb, codes8 = pl.pallas_call(
        functools.partial(_embed_body, BLK=EBLK, Cin=Cin, C=C, D=D, H=H, W=W),
        grid=(L // EBLK,),
        in_specs=[
            pl.BlockSpec((L + 2 * PAD, Cin), lambda i: (0, 0)),
            pl.BlockSpec((27, Cin, C), lambda i: (0, 0, 0)),
            pl.BlockSpec((Cin, Cin), lambda i: (0, 0)),
            pl.BlockSpec((C, 128), lambda i: (0, 0)),
            pl.BlockSpec((1, C), lambda i: (0, 0)),
            pl.BlockSpec((1, Cin), lambda i: (0, 0)),
        ],
        out_specs=[pl.BlockSpec((EBLK, 128), lambda i: (i, 0)),
                   pl.BlockSpec((EBLK, 8), lambda i: (i, 0))],
        out_shape=[jax.ShapeDtypeStruct((L, 128), jnp.float32),
                   jax.ShapeDtypeStruct((L, 8), jnp.int32)],
    )(xp, wt, wa, rotf, b_match.reshape(1, C), b_asm.reshape(1, Cin))

    RBLK = 1152
    NRB = L // RBLK
    NB = 64
    codes3 = codes8[:, :N_HASHES].T.reshape(N_HASHES, NRB, RBLK, 1)
    tril = jnp.tril(jnp.ones((RBLK, RBLK), jnp.float32), -1)
    within, counts = pl.pallas_call(
        functools.partial(_rank1_body, RBLK=RBLK, NB=NB),
        grid=(N_HASHES, NRB),
        in_specs=[
            pl.BlockSpec((1, 1, RBLK, 1), lambda h, rb: (h, rb, 0, 0)),
            pl.BlockSpec((RBLK, RBLK), lambda h, rb: (0, 0)),
        ],
        out_specs=[pl.BlockSpec((1, 1, RBLK, 1), lambda h, rb: (h, rb, 0, 0)),
                   pl.BlockSpec((1, 1, NB), lambda h, rb: (h, 0, 0))],
        out_shape=[jax.ShapeDtypeStruct((N_HASHES, NRB, RBLK, 1), jnp.float32),
                   jax.ShapeDtypeStruct((N_HASHES, 1, NB), jnp.float32)],
        scratch_shapes=[pltpu.VMEM((1, NB), jnp.float32)],
    )(codes3, tril)
    g4 = pl.pallas_call(
        functools.partial(_rank2_body, RBLK=RBLK, NB=NB, L=L),
        grid=(N_HASHES, NRB),
        in_specs=[
            pl.BlockSpec((1, 1, RBLK, 1), lambda h, rb: (h, rb, 0, 0)),
            pl.BlockSpec((1, 1, RBLK, 1), lambda h, rb: (h, rb, 0, 0)),
            pl.BlockSpec((1, 1, NB), lambda h, rb: (h, 0, 0)),
        ],
        out_specs=pl.BlockSpec((1, 1, RBLK, 1), lambda h, rb: (h, rb, 0, 0)),
        out_shape=jax.ShapeDtypeStruct((N_HASHES, NRB, RBLK, 1), jnp.int32),
    )(codes3, within, counts)
    g32 = g4.reshape(32, 9, 128)

    mesh = plsc.VectorSubcoreMesh(core_axis_name="c", subcore_axis_name="s")
    table = pl.kernel(
        functools.partial(_sc_scatter_body, NC=2, L=L),
        mesh=mesh,
        out_type=jax.ShapeDtypeStruct((N_HASHES * L, 128), jnp.float32),
        scratch_types=[pltpu.VMEM((9, 128), jnp.int32),
                       pltpu.VMEM((128, 128), jnp.float32),
                       pltpu.SemaphoreType.DMA],
    )(comb, g32)
    table = table.reshape(N_HASHES, NK, CHUNK, 128)

    GB = 8
    ret_plus = pl.pallas_call(
        functools.partial(_attn_body, C=C, Cin=Cin, GB=GB),
        grid=(N_HASHES, NK // GB),
        in_specs=[
            pl.BlockSpec((1, GB, CHUNK, 128), lambda h, kb: (h, kb, 0, 0)),
            pl.BlockSpec((1, 1, CHUNK, 128),
                         lambda h, kb: (h, (kb * GB - 1) % NK, 0, 0)),
            pl.BlockSpec((1, 1, CHUNK, 128),
                         lambda h, kb: (h, ((kb + 1) * GB) % NK, 0, 0)),
        ],
        out_specs=pl.BlockSpec((1, GB, CHUNK, 128), lambda h, kb: (h, kb, 0, 0)),
        out_shape=jax.ShapeDtypeStruct((N_HASHES, NK, CHUNK, 128), jnp.float32),
    )(table, table, table)

    rt = pl.kernel(
        functools.partial(_sc_gather_body, NC=2, L=L),
        mesh=mesh,
        out_type=jax.ShapeDtypeStruct((N_HASHES * L, 128), jnp.float32),
        scratch_types=[pltpu.VMEM((9, 128), jnp.int32),
                       pltpu.VMEM((128, 128), jnp.float32),
                       pltpu.SemaphoreType.DMA],
    )(ret_plus.reshape(N_HASHES * L, 128), g32)
    rt = rt.reshape(N_HASHES, L, 128)

    BLK = 1152
    PAD2 = 3 * BLK
    NCB = L // BLK
    xp1 = pl.pallas_call(
        functools.partial(_combine_body, Cin=Cin, NPB=3, NCB=NCB),
        grid=(NCB + 6,),
        in_specs=[
            pl.BlockSpec((N_HASHES, BLK, 128), lambda i: (0, (i - 3) % NCB, 0)),
            pl.BlockSpec((BLK, Cin), lambda i: ((i - 3) % NCB, 0)),
        ],
        out_specs=pl.BlockSpec((BLK, Cin), lambda i: (i, 0)),
        out_shape=jax.ShapeDtypeStruct((L + 2 * PAD2, Cin), jnp.float32),
    )(rt, xt)

    out = pl.pallas_call(
        functools.partial(_final_body, BLK=BLK, Cin=Cin, D=D, H=H, W=W, PADF=PAD2),
        grid=(L // BLK,),
        in_specs=[
            pl.BlockSpec((L + 2 * PAD2, Cin), lambda i: (0, 0)),
            pl.BlockSpec((27, Cin, Cin), lambda i: (0, 0, 0)),
            pl.BlockSpec((1, Cin), lambda i: (0, 0)),
        ],
        out_specs=pl.BlockSpec((BLK, Cin), lambda i: (i, 0)),
        out_shape=jax.ShapeDtypeStruct((L, Cin), jnp.float32),
    )(xp1, wc, b_conv.reshape(1, Cin))

    return out.T.reshape(N, Cin, D, H, W)


# speedup vs baseline: 1.1409x; 1.0205x over previous
import functools
import jax
import jax.numpy as jnp
from jax import lax
from jax.experimental import pallas as pl
from jax.experimental.pallas import tpu as pltpu
from jax.experimental.pallas import tpu_sc as plsc

N_HASHES = 4
CHUNK = 144
RED = 4
PAD = 2360


def _embed_body(xp_ref, wt_ref, wa_ref, rotf_ref, bm_ref, ba_ref, comb_ref, codes_ref,
                *, BLK, Cin, C, D, H, W):
    HW = H * W
    base = pl.program_id(0) * BLK
    t = base + lax.broadcasted_iota(jnp.int32, (BLK, 1), 0)
    w_out = t % W
    h_out = (t // W) % H
    acc = jnp.zeros((BLK, C), dtype=jnp.float32)
    for k in range(27):
        kz, ky, kx = k // 9, (k // 3) % 3, k % 3
        off = (kz - 1) * HW + (ky - 1) * W + (kx - 1)
        sl = xp_ref[pl.ds(PAD + off + base, BLK), :]
        wv = w_out + (kx - 1)
        hv = h_out + (ky - 1)
        valid = (wv >= 0) & (wv < W) & (hv >= 0) & (hv < H)
        sl = jnp.where(valid, sl, 0.0)
        acc = acc + jax.lax.dot_general(sl, wt_ref[k], (((1,), (0,)), ((), ())),
                                        preferred_element_type=jnp.float32)
    acc = acc + bm_ref[:]
    xc = xp_ref[pl.ds(PAD + base, BLK), :]
    ya = jax.lax.dot_general(xc, wa_ref[:], (((1,), (0,)), ((), ())),
                             preferred_element_type=jnp.float32) + ba_ref[:]
    pad = jnp.zeros((BLK, 128 - C - Cin), dtype=jnp.float32)
    comb_ref[:] = jnp.concatenate([acc, ya, pad], axis=1)
    r = jax.lax.dot_general(acc, rotf_ref[:], (((1,), (0,)), ((), ())),
                            preferred_element_type=jnp.float32)
    nb2 = rotf_ref.shape[1] // N_HASHES
    iota64 = lax.broadcasted_iota(jnp.int32, (BLK, 2 * nb2), 1)
    cols = []
    for h in range(N_HASHES):
        rh = r[:, nb2 * h:nb2 * (h + 1)]
        both = jnp.concatenate([rh, -rh], axis=1)
        mx = jnp.max(both, axis=1, keepdims=True)
        idx = jnp.min(jnp.where(both >= mx, iota64, 2 * nb2), axis=1)
        cols.append(idx[:, None].astype(jnp.int32))
    codes_ref[:] = jnp.concatenate(
        cols + [jnp.zeros((BLK, 8 - N_HASHES), jnp.int32)], axis=1)


def _rank1_body(codes_ref, tril_ref, within_ref, counts_ref, pref_ref, *, RBLK, NB):
    rb = pl.program_id(1)

    @pl.when(rb == 0)
    def _():
        pref_ref[:] = jnp.zeros((1, NB), jnp.float32)

    c = codes_ref[0, 0]
    onehot = (c == lax.broadcasted_iota(jnp.int32, (RBLK, NB), 1)).astype(jnp.float32)
    cum = jax.lax.dot_general(tril_ref[:], onehot, (((1,), (0,)), ((), ())),
                              preferred_element_type=jnp.float32) + pref_ref[:]
    within_ref[0, 0] = jnp.sum(cum * onehot, axis=1, keepdims=True)
    pref_ref[:] = pref_ref[:] + jnp.sum(onehot, axis=0, keepdims=True)
    counts_ref[0] = pref_ref[:]


def _rank2_body(codes_ref, within_ref, counts_ref, g_ref, *, RBLK, NB, L):
    h = pl.program_id(0)
    c = codes_ref[0, 0]
    onehot = (c == lax.broadcasted_iota(jnp.int32, (RBLK, NB), 1)).astype(jnp.float32)
    ci = lax.broadcasted_iota(jnp.int32, (NB, NB), 0)
    cj = lax.broadcasted_iota(jnp.int32, (NB, NB), 1)
    ut = (ci < cj).astype(jnp.float32)
    offs = jax.lax.dot_general(counts_ref[0], ut, (((1,), (0,)), ((), ())),
                               preferred_element_type=jnp.float32)
    offlook = jnp.sum(onehot * offs, axis=1, keepdims=True)
    g_ref[0, 0] = (within_ref[0, 0] + offlook).astype(jnp.int32) + h * L


def _attn_body(main_ref, left_ref, right_ref, out_ref, *, C, Cin, GB):
    left = left_ref[0, 0]
    right = right_ref[0, 0]
    for i in range(GB):
        cur = main_ref[0, i]
        prv = main_ref[0, i - 1] if i > 0 else left
        nxt = main_ref[0, i + 1] if i < GB - 1 else right
        xq = cur[:, :C]
        keys = jnp.concatenate([cur, prv, nxt], axis=0)
        xk = keys[:, :C]
        yv = keys[:, C:C + Cin]
        nrm = jnp.sqrt(jnp.sum(xk * xk, axis=1, keepdims=True))
        nrm = jnp.maximum(nrm, 5e-5)
        xkn = xk / nrm
        scores = jax.lax.dot_general(xq, xkn, (((1,), (1,)), ((), ())),
                                     preferred_element_type=jnp.float32)
        m = jnp.max(scores, axis=1, keepdims=True)
        e = jnp.exp(scores - m)
        s = jnp.sum(e, axis=1, keepdims=True)
        p = e / s
        ret = jax.lax.dot_general(p, yv, (((1,), (0,)), ((), ())),
                                  preferred_element_type=jnp.float32)
        b = m + jnp.log(s)
        out_ref[0, i] = jnp.concatenate(
            [ret, jnp.broadcast_to(b, (ret.shape[0], 128 - Cin))], axis=1)


def _combine_body(rt_ref, x_ref, out_ref, *, Cin, NPB, NCB):
    i = pl.program_id(0)
    out_ref[:] = jnp.zeros_like(out_ref)

    @pl.when((i >= NPB) & (i < NPB + NCB))
    def _():
        xs = [rt_ref[h, :, Cin:Cin + 1] for h in range(N_HASHES)]
        m = xs[0]
        for h in range(1, N_HASHES):
            m = jnp.maximum(m, xs[h])
        es = [jnp.exp(s - m) for s in xs]
        den = es[0]
        for h in range(1, N_HASHES):
            den = den + es[h]
        out = jnp.zeros_like(x_ref[:])
        for h in range(N_HASHES):
            out = out + rt_ref[h, :, :Cin] * (es[h] / den)
        out_ref[:] = out + x_ref[:]


def _final_body(xp_ref, wt_ref, bc_ref, out_ref, *, BLK, Cin, D, H, W, PADF):
    HW = H * W
    base = pl.program_id(0) * BLK
    t = base + lax.broadcasted_iota(jnp.int32, (BLK, 1), 0)
    w_out = t % W
    h_out = (t // W) % H
    acc = jnp.zeros((BLK, Cin), dtype=jnp.float32)
    for k in range(27):
        kz, ky, kx = k // 9, (k // 3) % 3, k % 3
        off = (kz - 1) * HW + (ky - 1) * W + (kx - 1)
        sl = jnp.maximum(xp_ref[pl.ds(PADF + off + base, BLK), :], 0.0)
        wv = w_out + (kx - 1)
        hv = h_out + (ky - 1)
        valid = (wv >= 0) & (wv < W) & (hv >= 0) & (hv < H)
        sl = jnp.where(valid, sl, 0.0)
        acc = acc + jax.lax.dot_general(sl, wt_ref[k], (((1,), (0,)), ((), ())),
                                        preferred_element_type=jnp.float32)
    out_ref[:] = xp_ref[pl.ds(PADF + base, BLK), :] + acc + bc_ref[:]


def _sc_scatter_body(comb_hbm, g_hbm, table_hbm, idx_v, src_v, sem, *, NC, L):
    wid = lax.axis_index("s") * NC + lax.axis_index("c")
    base_t = (wid % 8) * 1152
    pltpu.sync_copy(g_hbm.at[wid], idx_v)
    handles = []
    for j in range(9):
        b = j % 2
        if j >= 2:
            handles[j - 2].wait()
        pltpu.sync_copy(comb_hbm.at[pl.ds(base_t + j * 128, 128)], src_v.at[b])
        handles.append(pltpu.async_copy(src_v.at[b], table_hbm.at[idx_v.at[j]], sem))
    handles[7].wait()
    handles[8].wait()


def _sc_gather_body(table_hbm, g_hbm, out_hbm, idx_v, dst_v, sem, *, NC, L):
    wid = lax.axis_index("s") * NC + lax.axis_index("c")
    base = wid * 1152
    pltpu.sync_copy(g_hbm.at[wid], idx_v)
    handles = []
    for j in range(9):
        b = j % 2
        handles.append(pltpu.async_copy(table_hbm.at[idx_v.at[j]],
                                        dst_v.at[b], sem))
        if j >= 1:
            handles[j - 1].wait()
            pltpu.sync_copy(dst_v.at[(j - 1) % 2],
                            out_hbm.at[pl.ds(base + (j - 1) * 128, 128)])
    handles[8].wait()
    pltpu.sync_copy(dst_v.at[0], out_hbm.at[pl.ds(base + 8 * 128, 128)])


def kernel(x, w_match, b_match, w_asm, b_asm, rot, w_conv, b_conv):
    N, Cin, D, H, W = x.shape
    L = D * H * W
    C = Cin // RED
    NK = L // CHUNK
    xt = x.reshape(Cin, L).T
    xp = jnp.zeros((L + 2 * PAD, Cin), jnp.float32).at[PAD:PAD + L].set(xt)
    wt = w_match.transpose(2, 3, 4, 1, 0).reshape(27, Cin, C)
    wa = w_asm.reshape(Cin, Cin).T
    rotf = rot.reshape(C, -1)
    wc = w_conv.transpose(2, 3, 4, 1, 0).reshape(27, Cin, Cin)

    EBLK = 1152
    comb, codes8 = pl.pallas_call(
        functools.partial(_embed_body, BLK=EBLK, Cin=Cin, C=C, D=D, H=H, W=W),
        grid=(L // EBLK,),
        in_specs=[
            pl.BlockSpec((L + 2 * PAD, Cin), lambda i: (0, 0)),
            pl.BlockSpec((27, Cin, C), lambda i: (0, 0, 0)),
            pl.BlockSpec((Cin, Cin), lambda i: (0, 0)),
            pl.BlockSpec((C, 128), lambda i: (0, 0)),
            pl.BlockSpec((1, C), lambda i: (0, 0)),
            pl.BlockSpec((1, Cin), lambda i: (0, 0)),
        ],
        out_specs=[pl.BlockSpec((EBLK, 128), lambda i: (i, 0)),
                   pl.BlockSpec((EBLK, 8), lambda i: (i, 0))],
        out_shape=[jax.ShapeDtypeStruct((L, 128), jnp.float32),
                   jax.ShapeDtypeStruct((L, 8), jnp.int32)],
    )(xp, wt, wa, rotf, b_match.reshape(1, C), b_asm.reshape(1, Cin))

    RBLK = 1152
    NRB = L // RBLK
    NB = 64
    codes3 = codes8[:, :N_HASHES].T.reshape(N_HASHES, NRB, RBLK, 1)
    tril = jnp.tril(jnp.ones((RBLK, RBLK), jnp.float32), -1)
    within, counts = pl.pallas_call(
        functools.partial(_rank1_body, RBLK=RBLK, NB=NB),
        grid=(N_HASHES, NRB),
        in_specs=[
            pl.BlockSpec((1, 1, RBLK, 1), lambda h, rb: (h, rb, 0, 0)),
            pl.BlockSpec((RBLK, RBLK), lambda h, rb: (0, 0)),
        ],
        out_specs=[pl.BlockSpec((1, 1, RBLK, 1), lambda h, rb: (h, rb, 0, 0)),
                   pl.BlockSpec((1, 1, NB), lambda h, rb: (h, 0, 0))],
        out_shape=[jax.ShapeDtypeStruct((N_HASHES, NRB, RBLK, 1), jnp.float32),
                   jax.ShapeDtypeStruct((N_HASHES, 1, NB), jnp.float32)],
        scratch_shapes=[pltpu.VMEM((1, NB), jnp.float32)],
    )(codes3, tril)
    g4 = pl.pallas_call(
        functools.partial(_rank2_body, RBLK=RBLK, NB=NB, L=L),
        grid=(N_HASHES, NRB),
        in_specs=[
            pl.BlockSpec((1, 1, RBLK, 1), lambda h, rb: (h, rb, 0, 0)),
            pl.BlockSpec((1, 1, RBLK, 1), lambda h, rb: (h, rb, 0, 0)),
            pl.BlockSpec((1, 1, NB), lambda h, rb: (h, 0, 0)),
        ],
        out_specs=pl.BlockSpec((1, 1, RBLK, 1), lambda h, rb: (h, rb, 0, 0)),
        out_shape=jax.ShapeDtypeStruct((N_HASHES, NRB, RBLK, 1), jnp.int32),
    )(codes3, within, counts)
    g32 = g4.reshape(32, 9, 128)

    mesh = plsc.VectorSubcoreMesh(core_axis_name="c", subcore_axis_name="s")
    table = pl.kernel(
        functools.partial(_sc_scatter_body, NC=2, L=L),
        mesh=mesh,
        out_type=jax.ShapeDtypeStruct((N_HASHES * L, 128), jnp.float32),
        scratch_types=[pltpu.VMEM((9, 128), jnp.int32),
                       pltpu.VMEM((2, 128, 128), jnp.float32),
                       pltpu.SemaphoreType.DMA],
    )(comb, g32)
    table = table.reshape(N_HASHES, NK, CHUNK, 128)

    GB = 8
    ret_plus = pl.pallas_call(
        functools.partial(_attn_body, C=C, Cin=Cin, GB=GB),
        grid=(N_HASHES, NK // GB),
        in_specs=[
            pl.BlockSpec((1, GB, CHUNK, 128), lambda h, kb: (h, kb, 0, 0)),
            pl.BlockSpec((1, 1, CHUNK, 128),
                         lambda h, kb: (h, (kb * GB - 1) % NK, 0, 0)),
            pl.BlockSpec((1, 1, CHUNK, 128),
                         lambda h, kb: (h, ((kb + 1) * GB) % NK, 0, 0)),
        ],
        out_specs=pl.BlockSpec((1, GB, CHUNK, 128), lambda h, kb: (h, kb, 0, 0)),
        out_shape=jax.ShapeDtypeStruct((N_HASHES, NK, CHUNK, 128), jnp.float32),
    )(table, table, table)

    rt = pl.kernel(
        functools.partial(_sc_gather_body, NC=2, L=L),
        mesh=mesh,
        out_type=jax.ShapeDtypeStruct((N_HASHES * L, 128), jnp.float32),
        scratch_types=[pltpu.VMEM((9, 128), jnp.int32),
                       pltpu.VMEM((2, 128, 128), jnp.float32),
                       pltpu.SemaphoreType.DMA],
    )(ret_plus.reshape(N_HASHES * L, 128), g32)
    rt = rt.reshape(N_HASHES, L, 128)

    BLK = 1152
    PAD2 = 3 * BLK
    NCB = L // BLK
    xp1 = pl.pallas_call(
        functools.partial(_combine_body, Cin=Cin, NPB=3, NCB=NCB),
        grid=(NCB + 6,),
        in_specs=[
            pl.BlockSpec((N_HASHES, BLK, 128), lambda i: (0, (i - 3) % NCB, 0)),
            pl.BlockSpec((BLK, Cin), lambda i: ((i - 3) % NCB, 0)),
        ],
        out_specs=pl.BlockSpec((BLK, Cin), lambda i: (i, 0)),
        out_shape=jax.ShapeDtypeStruct((L + 2 * PAD2, Cin), jnp.float32),
    )(rt, xt)

    out = pl.pallas_call(
        functools.partial(_final_body, BLK=BLK, Cin=Cin, D=D, H=H, W=W, PADF=PAD2),
        grid=(L // BLK,),
        in_specs=[
            pl.BlockSpec((L + 2 * PAD2, Cin), lambda i: (0, 0)),
            pl.BlockSpec((27, Cin, Cin), lambda i: (0, 0, 0)),
            pl.BlockSpec((1, Cin), lambda i: (0, 0)),
        ],
        out_specs=pl.BlockSpec((BLK, Cin), lambda i: (i, 0)),
        out_shape=jax.ShapeDtypeStruct((L, Cin), jnp.float32),
    )(xp1, wc, b_conv.reshape(1, Cin))

    return out.T.reshape(N, Cin, D, H, W)
